# baseline (speedup 1.0000x reference)
import jax
import jax.numpy as jnp
from jax.experimental import pallas as pl

_BN = 8192


def _body(a_ref, b_ref, xui_ref):
    xui_ref[...] = jnp.sum(a_ref[...] * b_ref[...], axis=0)


def kernel(gum, gim):
    n_rows, n_cols = gum.shape
    a = gum.T
    b = gim.T
    grid = (n_rows // _BN,)
    xui = pl.pallas_call(
        _body,
        grid=grid,
        in_specs=[
            pl.BlockSpec((n_cols, _BN), lambda i: (0, i)),
            pl.BlockSpec((n_cols, _BN), lambda i: (0, i)),
        ],
        out_specs=pl.BlockSpec((_BN,), lambda i: (i,)),
        out_shape=jax.ShapeDtypeStruct((n_rows,), jnp.float32),
    )(a, b)
    return (xui, gum, gim)


# speedup vs baseline: 1.5649x; 1.5649x over previous
import jax
import jax.numpy as jnp
from jax.experimental import pallas as pl
from jax.experimental.pallas import tpu as pltpu

_CH = 2048


def _body(a, b, xui, ao, bo, av, bv, xv, in_sems, out_sems, xsem):
    n = a.shape[1] // _CH
    for c in range(n):
        sl = pl.ds(c * _CH, _CH)
        pltpu.make_async_copy(a.at[:, sl], av.at[:, sl], in_sems.at[c, 0]).start()
        pltpu.make_async_copy(b.at[:, sl], bv.at[:, sl], in_sems.at[c, 1]).start()
    for c in range(n):
        sl = pl.ds(c * _CH, _CH)
        pltpu.make_async_copy(a.at[:, sl], av.at[:, sl], in_sems.at[c, 0]).wait()
        pltpu.make_async_copy(b.at[:, sl], bv.at[:, sl], in_sems.at[c, 1]).wait()
        xv[sl] = jnp.sum(av[:, sl] * bv[:, sl], axis=0)
        pltpu.make_async_copy(av.at[:, sl], ao.at[:, sl], out_sems.at[c, 0]).start()
        pltpu.make_async_copy(bv.at[:, sl], bo.at[:, sl], out_sems.at[c, 1]).start()
    pltpu.make_async_copy(xv, xui, xsem).start()
    for c in range(n):
        sl = pl.ds(c * _CH, _CH)
        pltpu.make_async_copy(av.at[:, sl], ao.at[:, sl], out_sems.at[c, 0]).wait()
        pltpu.make_async_copy(bv.at[:, sl], bo.at[:, sl], out_sems.at[c, 1]).wait()
    pltpu.make_async_copy(xv, xui, xsem).wait()


def kernel(gum, gim):
    n_rows, n_cols = gum.shape
    a = gum.T
    b = gim.T
    n_chunks = n_rows // _CH
    xui, a_o, b_o = pl.pallas_call(
        _body,
        in_specs=[
            pl.BlockSpec(memory_space=pl.ANY),
            pl.BlockSpec(memory_space=pl.ANY),
        ],
        out_specs=[
            pl.BlockSpec(memory_space=pl.ANY),
            pl.BlockSpec(memory_space=pl.ANY),
            pl.BlockSpec(memory_space=pl.ANY),
        ],
        out_shape=[
            jax.ShapeDtypeStruct((n_rows,), jnp.float32),
            jax.ShapeDtypeStruct((n_cols, n_rows), jnp.float32),
            jax.ShapeDtypeStruct((n_cols, n_rows), jnp.float32),
        ],
        scratch_shapes=[
            pltpu.VMEM((n_cols, n_rows), jnp.float32),
            pltpu.VMEM((n_cols, n_rows), jnp.float32),
            pltpu.VMEM((n_rows,), jnp.float32),
            pltpu.SemaphoreType.DMA((n_chunks, 2)),
            pltpu.SemaphoreType.DMA((n_chunks, 2)),
            pltpu.SemaphoreType.DMA,
        ],
    )(a, b)
    return (xui, a_o.T, b_o.T)
